# baseline (speedup 1.0000x reference)
import functools

import jax
import jax.numpy as jnp
from jax import lax
from jax.experimental import pallas as pl
from jax.experimental.pallas import tpu as pltpu
from jax.experimental.pallas import tpu_sc as plsc

_IDX_CHUNK = 128


@functools.lru_cache(maxsize=None)
def _build(V, D, B, SEQ):
    info = plsc.get_sparse_core_info()
    NC, NS, L = info.num_cores, info.num_subcores, info.num_lanes
    NW = NC * NS
    assert B % NW == 0
    b_per_w = B // NW
    assert b_per_w % _IDX_CHUNK == 0
    n_chunks = b_per_w // _IDX_CHUNK
    assert SEQ % b_per_w == 0 and D % L == 0
    scale = float(D) ** 0.5
    mesh = plsc.VectorSubcoreMesh(core_axis_name="c", subcore_axis_name="s")

    @functools.partial(
        pl.kernel,
        mesh=mesh,
        out_type=jax.ShapeDtypeStruct((B, D), jnp.float32),
        scratch_types=[
            pltpu.VMEM((n_chunks, _IDX_CHUNK), jnp.int32),
            pltpu.VMEM((b_per_w, D), jnp.float32),
            pltpu.VMEM((b_per_w, D), jnp.float32),
            pltpu.SemaphoreType.DMA,
        ],
    )
    def emb_kernel(idx_hbm, table_hbm, pe_hbm, out_hbm, idx_v, rows_v, pe_v, sem):
        c = lax.axis_index("c")
        s = lax.axis_index("s")
        wid = s * NC + c
        base = wid * b_per_w
        pe_base = lax.rem(base, SEQ)

        pltpu.sync_copy(idx_hbm.at[pl.ds(wid * n_chunks, n_chunks)], idx_v)

        copies = [
            pltpu.async_copy(
                table_hbm.at[idx_v.at[j]],
                rows_v.at[pl.ds(j * _IDX_CHUNK, _IDX_CHUNK)],
                sem,
            )
            for j in range(n_chunks)
        ]
        pltpu.sync_copy(pe_hbm.at[pl.ds(pe_base, b_per_w)], pe_v)
        for cp in copies:
            cp.wait()

        def body(i, carry):
            for j in range(D // L):
                sl = pl.ds(j * L, L)
                rows_v[i, sl] = rows_v[i, sl] * scale + pe_v[i, sl]
            return carry

        lax.fori_loop(0, b_per_w, body, 0, unroll=4)

        pltpu.sync_copy(rows_v, out_hbm.at[pl.ds(base, b_per_w)])

    return emb_kernel


def kernel(src_ids, W, pe):
    BATCH, SEQ = src_ids.shape
    V, D = W.shape
    B = BATCH * SEQ
    idx = src_ids.reshape(-1).astype(jnp.int32).reshape(B // _IDX_CHUNK, _IDX_CHUNK)
    out = _build(V, D, B, SEQ)(idx, W, pe)
    return out.reshape(BATCH, SEQ, D)


# speedup vs baseline: 1.0933x; 1.0933x over previous
import functools

import jax
import jax.numpy as jnp
from jax import lax
from jax.experimental import pallas as pl
from jax.experimental.pallas import tpu as pltpu
from jax.experimental.pallas import tpu_sc as plsc

_CHUNK = 64


@functools.lru_cache(maxsize=None)
def _build(V, D, B, SEQ):
    info = plsc.get_sparse_core_info()
    NC, NS, L = info.num_cores, info.num_subcores, info.num_lanes
    NW = NC * NS
    assert B % NW == 0
    b_per_w = B // NW
    assert b_per_w % _CHUNK == 0
    n_chunks = b_per_w // _CHUNK
    assert SEQ % b_per_w == 0 and D % L == 0
    scale = float(D) ** 0.5
    mesh = plsc.VectorSubcoreMesh(core_axis_name="c", subcore_axis_name="s")

    @functools.partial(
        pl.kernel,
        mesh=mesh,
        out_type=jax.ShapeDtypeStruct((B, D), jnp.float32),
        scratch_types=[
            pltpu.VMEM((n_chunks, _CHUNK), jnp.int32),
            pltpu.VMEM((b_per_w, D), jnp.float32),
            pltpu.VMEM((b_per_w, D), jnp.float32),
            pltpu.SemaphoreType.DMA,
            pltpu.SemaphoreType.DMA,
            pltpu.SemaphoreType.DMA,
        ],
    )
    def emb_kernel(idx_hbm, table_hbm, pe_hbm, out_hbm, idx_v, rows_v, pe_v,
                   sem_g, sem_pe, sem_w):
        c = lax.axis_index("c")
        s = lax.axis_index("s")
        wid = s * NC + c
        base = wid * b_per_w
        pe_base = lax.rem(base, SEQ)

        pltpu.sync_copy(idx_hbm.at[pl.ds(wid * n_chunks, n_chunks)], idx_v)

        pe_cp = pltpu.async_copy(pe_hbm.at[pl.ds(pe_base, b_per_w)], pe_v, sem_pe)
        gathers = [
            pltpu.async_copy(
                table_hbm.at[idx_v.at[k]],
                rows_v.at[pl.ds(k * _CHUNK, _CHUNK)],
                sem_g,
            )
            for k in range(n_chunks)
        ]
        pe_cp.wait()

        writes = []
        for k in range(n_chunks):
            gathers[k].wait()

            def body(i, carry):
                for j in range(D // L):
                    sl = pl.ds(j * L, L)
                    rows_v[i, sl] = rows_v[i, sl] * scale + pe_v[i, sl]
                return carry

            lax.fori_loop(k * _CHUNK, (k + 1) * _CHUNK, body, 0, unroll=4)
            writes.append(
                pltpu.async_copy(
                    rows_v.at[pl.ds(k * _CHUNK, _CHUNK)],
                    out_hbm.at[pl.ds(base + k * _CHUNK, _CHUNK)],
                    sem_w,
                )
            )
        for wcp in writes:
            wcp.wait()

    return emb_kernel


def kernel(src_ids, W, pe):
    BATCH, SEQ = src_ids.shape
    V, D = W.shape
    B = BATCH * SEQ
    idx = src_ids.reshape(-1).astype(jnp.int32).reshape(B // _CHUNK, _CHUNK)
    out = _build(V, D, B, SEQ)(idx, W, pe)
    return out.reshape(BATCH, SEQ, D)


# speedup vs baseline: 1.2550x; 1.1478x over previous
import functools

import jax
import jax.numpy as jnp
from jax import lax
from jax.experimental import pallas as pl
from jax.experimental.pallas import tpu as pltpu
from jax.experimental.pallas import tpu_sc as plsc

_CHUNK = 64


@functools.lru_cache(maxsize=None)
def _build(V, D, B, SEQ):
    info = plsc.get_sparse_core_info()
    NC, NS, L = info.num_cores, info.num_subcores, info.num_lanes
    NW = NC * NS
    assert B % NW == 0
    b_per_w = B // NW
    assert b_per_w % _CHUNK == 0
    n_chunks = b_per_w // _CHUNK
    assert SEQ % b_per_w == 0 and D % L == 0
    scale = float(D) ** 0.5
    mesh = plsc.VectorSubcoreMesh(core_axis_name="c", subcore_axis_name="s")

    @functools.partial(
        pl.kernel,
        mesh=mesh,
        out_type=jax.ShapeDtypeStruct((B, D), jnp.float32),
        scratch_types=[
            pltpu.VMEM((n_chunks, _CHUNK), jnp.int32),
            pltpu.VMEM((b_per_w, D), jnp.float32),
            pltpu.VMEM((b_per_w, D), jnp.float32),
            pltpu.SemaphoreType.DMA,
            pltpu.SemaphoreType.DMA,
            pltpu.SemaphoreType.DMA,
        ],
    )
    def emb_kernel(idx_hbm, table_hbm, pe_hbm, out_hbm, idx_v, rows_v, pe_v,
                   sem_g, sem_pe, sem_w):
        c = lax.axis_index("c")
        s = lax.axis_index("s")
        wid = s * NC + c
        base = wid * b_per_w
        pe_base = lax.rem(base, SEQ)

        pltpu.sync_copy(idx_hbm.at[pl.ds(wid * n_chunks, n_chunks)], idx_v)

        pe_cp = pltpu.async_copy(pe_hbm.at[pl.ds(pe_base, b_per_w)], pe_v, sem_pe)
        gathers = [
            pltpu.async_copy(
                table_hbm.at[idx_v.at[k]],
                rows_v.at[pl.ds(k * _CHUNK, _CHUNK)],
                sem_g,
            )
            for k in range(n_chunks)
        ]
        pe_cp.wait()

        writes = []
        for k in range(n_chunks):
            gathers[k].wait()

            @plsc.parallel_loop(k * _CHUNK, (k + 1) * _CHUNK, step=1, unroll=4)
            def body(i):
                for j in range(D // L):
                    sl = pl.ds(j * L, L)
                    rows_v[i, sl] = rows_v[i, sl] * scale + pe_v[i, sl]
            writes.append(
                pltpu.async_copy(
                    rows_v.at[pl.ds(k * _CHUNK, _CHUNK)],
                    out_hbm.at[pl.ds(base + k * _CHUNK, _CHUNK)],
                    sem_w,
                )
            )
        for wcp in writes:
            wcp.wait()

    return emb_kernel


def kernel(src_ids, W, pe):
    BATCH, SEQ = src_ids.shape
    V, D = W.shape
    B = BATCH * SEQ
    idx = src_ids.reshape(-1).astype(jnp.int32).reshape(B // _CHUNK, _CHUNK)
    out = _build(V, D, B, SEQ)(idx, W, pe)
    return out.reshape(BATCH, SEQ, D)


# speedup vs baseline: 1.3685x; 1.0905x over previous
import functools

import jax
import jax.numpy as jnp
from jax import lax
from jax.experimental import pallas as pl
from jax.experimental.pallas import tpu as pltpu
from jax.experimental.pallas import tpu_sc as plsc

_CHUNK = 64


@functools.lru_cache(maxsize=None)
def _build(V, D, BATCH, SEQ):
    info = plsc.get_sparse_core_info()
    NC, NS, L = info.num_cores, info.num_subcores, info.num_lanes
    NW = NC * NS
    B = BATCH * SEQ
    assert SEQ == NW * _CHUNK and D % L == 0
    rows_per_w = BATCH * _CHUNK
    scale = float(D) ** 0.5
    mesh = plsc.VectorSubcoreMesh(core_axis_name="c", subcore_axis_name="s")

    @functools.partial(
        pl.kernel,
        mesh=mesh,
        out_type=jax.ShapeDtypeStruct((B, D), jnp.float32),
        scratch_types=[
            pltpu.VMEM((BATCH, _CHUNK), jnp.int32),
            pltpu.VMEM((rows_per_w, D), jnp.float32),
            pltpu.VMEM((_CHUNK, D), jnp.float32),
            pltpu.SemaphoreType.DMA,
            pltpu.SemaphoreType.DMA,
            pltpu.SemaphoreType.DMA,
            pltpu.SemaphoreType.DMA,
        ],
    )
    def emb_kernel(idx_hbm, table_hbm, pe_hbm, out_hbm, idx_v, rows_v, pe_v,
                   sem_i, sem_g, sem_pe, sem_w):
        c = lax.axis_index("c")
        s = lax.axis_index("s")
        t = s * NC + c

        pe_cp = pltpu.async_copy(pe_hbm.at[pl.ds(t * _CHUNK, _CHUNK)], pe_v, sem_pe)
        idx_cps = [
            pltpu.async_copy(idx_hbm.at[b * NW + t], idx_v.at[b], sem_i)
            for b in range(BATCH)
        ]
        for cp in idx_cps:
            cp.wait()
        gathers = [
            pltpu.async_copy(
                table_hbm.at[idx_v.at[b]],
                rows_v.at[pl.ds(b * _CHUNK, _CHUNK)],
                sem_g,
            )
            for b in range(BATCH)
        ]
        pe_cp.wait()

        writes = []
        for b in range(BATCH):
            gathers[b].wait()

            @plsc.parallel_loop(0, _CHUNK, step=1, unroll=4)
            def body(i):
                for j in range(D // L):
                    sl = pl.ds(j * L, L)
                    rows_v[b * _CHUNK + i, sl] = (
                        rows_v[b * _CHUNK + i, sl] * scale + pe_v[i, sl]
                    )

            writes.append(
                pltpu.async_copy(
                    rows_v.at[pl.ds(b * _CHUNK, _CHUNK)],
                    out_hbm.at[pl.ds(b * SEQ + t * _CHUNK, _CHUNK)],
                    sem_w,
                )
            )
        for wcp in writes:
            wcp.wait()

    return emb_kernel


def kernel(src_ids, W, pe):
    BATCH, SEQ = src_ids.shape
    V, D = W.shape
    idx = src_ids.reshape(-1).astype(jnp.int32).reshape(-1, _CHUNK)
    out = _build(V, D, BATCH, SEQ)(idx, W, pe)
    return out.reshape(BATCH, SEQ, D)


# speedup vs baseline: 1.3724x; 1.0028x over previous
import functools

import jax
import jax.numpy as jnp
from jax import lax
from jax.experimental import pallas as pl
from jax.experimental.pallas import tpu as pltpu
from jax.experimental.pallas import tpu_sc as plsc

_CHUNK = 64


@functools.lru_cache(maxsize=None)
def _build(V, D, BATCH, SEQ):
    info = plsc.get_sparse_core_info()
    NC, NS, L = info.num_cores, info.num_subcores, info.num_lanes
    NW = NC * NS
    B = BATCH * SEQ
    assert SEQ == NW * _CHUNK and D % L == 0
    rows_per_w = BATCH * _CHUNK
    scale = float(D) ** 0.5
    mesh = plsc.VectorSubcoreMesh(core_axis_name="c", subcore_axis_name="s")

    @functools.partial(
        pl.kernel,
        mesh=mesh,
        out_type=jax.ShapeDtypeStruct((BATCH, SEQ, D), jnp.float32),
        scratch_types=[
            pltpu.VMEM((BATCH, _CHUNK), jnp.int32),
            pltpu.VMEM((rows_per_w, D), jnp.float32),
            pltpu.VMEM((_CHUNK, D), jnp.float32),
            pltpu.SemaphoreType.DMA,
            pltpu.SemaphoreType.DMA,
            pltpu.SemaphoreType.DMA,
            pltpu.SemaphoreType.DMA,
        ],
    )
    def emb_kernel(idx_hbm, table_hbm, pe_hbm, out_hbm, idx_v, rows_v, pe_v,
                   sem_i, sem_g, sem_pe, sem_w):
        c = lax.axis_index("c")
        s = lax.axis_index("s")
        t = s * NC + c

        pe_cp = pltpu.async_copy(pe_hbm.at[pl.ds(t * _CHUNK, _CHUNK)], pe_v, sem_pe)
        idx_cps = [
            pltpu.async_copy(idx_hbm.at[b, pl.ds(t * _CHUNK, _CHUNK)], idx_v.at[b], sem_i)
            for b in range(BATCH)
        ]
        for cp in idx_cps:
            cp.wait()
        gathers = [
            pltpu.async_copy(
                table_hbm.at[idx_v.at[b]],
                rows_v.at[pl.ds(b * _CHUNK, _CHUNK)],
                sem_g,
            )
            for b in range(BATCH)
        ]
        pe_cp.wait()

        writes = []
        for b in range(BATCH):
            gathers[b].wait()

            @plsc.parallel_loop(0, _CHUNK, step=1, unroll=4)
            def body(i):
                for j in range(D // L):
                    sl = pl.ds(j * L, L)
                    rows_v[b * _CHUNK + i, sl] = (
                        rows_v[b * _CHUNK + i, sl] * scale + pe_v[i, sl]
                    )

            writes.append(
                pltpu.async_copy(
                    rows_v.at[pl.ds(b * _CHUNK, _CHUNK)],
                    out_hbm.at[b, pl.ds(t * _CHUNK, _CHUNK)],
                    sem_w,
                )
            )
        for wcp in writes:
            wcp.wait()

    return emb_kernel


def kernel(src_ids, W, pe):
    BATCH, SEQ = src_ids.shape
    V, D = W.shape
    return _build(V, D, BATCH, SEQ)(src_ids.astype(jnp.int32), W, pe)
